# Initial kernel scaffold; baseline (speedup 1.0000x reference)
import jax
import jax.numpy as jnp
from jax.experimental import pallas as pl


def kernel(node_input, node_attr, edge_src, edge_dst, edge_attr, edge_scalars, W_sc, W_lin1, W_lin2, W_fc1, W_fc2):
    raise NotImplementedError("write your pallas kernel here")


# speedup vs baseline: 2.1132x; 2.1132x over previous
import functools
import math

import jax
import jax.numpy as jnp
from jax import lax
from jax.experimental import pallas as pl
from jax.experimental.pallas import tpu as pltpu
from jax.experimental.pallas import tpu_sc as plsc

NC = 2
NS = 16
NW = NC * NS
LANES = 16
CHUNK = 128
LOG2 = math.log(2.0)
AVG_DEG = 32.0
BE = 2048
BN = 1000


def _ssp(x):
    return jnp.maximum(x, 0.0) + jnp.log(1.0 + jnp.exp(-jnp.abs(x))) - LOG2


def _edge_w_body(es_ref, ea_ref, w1_ref, w2_ref, out_ref):
    h = jnp.dot(es_ref[...], w1_ref[...], preferred_element_type=jnp.float32)
    w = jnp.dot(_ssp(h), w2_ref[...], preferred_element_type=jnp.float32)
    out_ref[...] = w * ea_ref[...]


def _edge_w(es, ea, W1, W2):
    E_pad, DE = es.shape
    H, D = W2.shape
    return pl.pallas_call(
        _edge_w_body,
        grid=(E_pad // BE,),
        in_specs=[
            pl.BlockSpec((BE, DE), lambda e: (e, 0)),
            pl.BlockSpec((BE, 1), lambda e: (e, 0)),
            pl.BlockSpec((DE, H), lambda e: (0, 0)),
            pl.BlockSpec((H, D), lambda e: (0, 0)),
        ],
        out_specs=pl.BlockSpec((BE, D), lambda e: (e, 0)),
        out_shape=jax.ShapeDtypeStruct((E_pad, D), jnp.float32),
    )(es, ea, W1, W2)


def _node_mm_body(x_ref, attr_ref, wsc_ref, wl1_ref, sc_ref, feat_ref):
    y = x_ref[...] * attr_ref[...]
    sc_ref[...] = jnp.dot(y, wsc_ref[...], preferred_element_type=jnp.float32)
    feat_ref[...] = jnp.dot(y, wl1_ref[...], preferred_element_type=jnp.float32)


def _node_mm(x, attr, Wsc, Wl1):
    N, D = x.shape
    return pl.pallas_call(
        _node_mm_body,
        grid=(N // BN,),
        in_specs=[
            pl.BlockSpec((BN, D), lambda n: (n, 0)),
            pl.BlockSpec((BN, 1), lambda n: (n, 0)),
            pl.BlockSpec((D, D), lambda n: (0, 0)),
            pl.BlockSpec((D, D), lambda n: (0, 0)),
        ],
        out_specs=[
            pl.BlockSpec((BN, D), lambda n: (n, 0)),
            pl.BlockSpec((BN, D), lambda n: (n, 0)),
        ],
        out_shape=[
            jax.ShapeDtypeStruct((N, D), jnp.float32),
            jax.ShapeDtypeStruct((N, D), jnp.float32),
        ],
    )(x, attr, Wsc, Wl1)


def _update_body(p_ref, sc_ref, attr_ref, wl2_ref, x_ref):
    agg = (p_ref[0] + p_ref[1]) * (1.0 / math.sqrt(AVG_DEG))
    conv = jnp.dot(agg * attr_ref[...], wl2_ref[...],
                   preferred_element_type=jnp.float32)
    x_ref[...] = _ssp(sc_ref[...] + conv)


def _update(parts, sc, attr, Wl2):
    N, D = sc.shape
    return pl.pallas_call(
        _update_body,
        grid=(N // BN,),
        in_specs=[
            pl.BlockSpec((NC, BN, D), lambda n: (0, n, 0)),
            pl.BlockSpec((BN, D), lambda n: (n, 0)),
            pl.BlockSpec((BN, 1), lambda n: (n, 0)),
            pl.BlockSpec((D, D), lambda n: (0, 0)),
        ],
        out_specs=pl.BlockSpec((BN, D), lambda n: (n, 0)),
        out_shape=jax.ShapeDtypeStruct((N, D), jnp.float32),
    )(parts, sc, attr, Wl2)


def _sc_gather_scatter(feat, w_e, src, dst, zeros):
    N, D = feat.shape
    E_pad = src.shape[0]
    N_pad = zeros.shape[0]
    EPW = E_pad // NW
    n_chunks = EPW // CHUNK
    RPS = N_pad // NS
    mesh = plsc.VectorSubcoreMesh(core_axis_name="c", subcore_axis_name="s")

    @functools.partial(
        pl.kernel,
        out_type=jax.ShapeDtypeStruct((NC, N_pad, D), jnp.float32),
        mesh=mesh,
        scratch_types=[
            pltpu.VMEM_SHARED((N_pad, D), jnp.float32),
            pltpu.VMEM((CHUNK,), jnp.int32),
            pltpu.VMEM((CHUNK,), jnp.int32),
            pltpu.VMEM((CHUNK, D), jnp.float32),
            pltpu.VMEM((CHUNK, D), jnp.float32),
            pltpu.SemaphoreType.DMA,
        ],
    )
    def k(feat_hbm, w_hbm, src_hbm, dst_hbm, zeros_hbm, out_hbm,
          acc_sh, sidx_v, didx_v, rows_v, wrow_v, sem):
        c = lax.axis_index("c")
        s = lax.axis_index("s")
        wid = s * NC + c
        pltpu.sync_copy(zeros_hbm.at[pl.ds(s * RPS, RPS)],
                        acc_sh.at[pl.ds(s * RPS, RPS)])
        plsc.subcore_barrier()
        base0 = wid * EPW

        def chunk_body(kk, carry):
            base = base0 + kk * CHUNK
            pltpu.sync_copy(src_hbm.at[pl.ds(base, CHUNK)], sidx_v)
            pltpu.sync_copy(dst_hbm.at[pl.ds(base, CHUNK)], didx_v)
            pltpu.async_copy(feat_hbm.at[sidx_v], rows_v, sem).wait()
            pltpu.sync_copy(w_hbm.at[pl.ds(base, CHUNK)], wrow_v)

            def row_body(r, cc):
                for db in range(D // LANES):
                    sl = pl.ds(db * LANES, LANES)
                    rows_v[r, sl] = rows_v[r, sl] * wrow_v[r, sl]
                return cc

            lax.fori_loop(0, CHUNK, row_body, 0)
            pltpu.sync_copy(rows_v, acc_sh.at[didx_v], add=True)
            return carry

        lax.fori_loop(0, n_chunks, chunk_body, 0)
        plsc.subcore_barrier()
        pltpu.sync_copy(acc_sh.at[pl.ds(s * RPS, RPS)],
                        out_hbm.at[c, pl.ds(s * RPS, RPS)])

    return k(feat, w_e, src, dst, zeros)


def kernel(node_input, node_attr, edge_src, edge_dst, edge_attr, edge_scalars,
           W_sc, W_lin1, W_lin2, W_fc1, W_fc2):
    N, D = node_input.shape
    E = edge_src.shape[0]
    L = W_sc.shape[0]
    EPW = ((E + NW - 1) // NW + CHUNK - 1) // CHUNK * CHUNK
    E_pad = EPW * NW
    pe = E_pad - E
    src = jnp.pad(edge_src.astype(jnp.int32), (0, pe))
    dst = jnp.pad(edge_dst.astype(jnp.int32), (0, pe))
    ea = jnp.pad(edge_attr.astype(jnp.float32), ((0, pe), (0, 0)))
    es = jnp.pad(edge_scalars.astype(jnp.float32), ((0, pe), (0, 0)))
    N_pad = (N + 8 * NS - 1) // (8 * NS) * (8 * NS)
    zeros = jnp.zeros((N_pad, D), jnp.float32)

    w_layers = [_edge_w(es, ea, W_fc1[i], W_fc2[i]) for i in range(L)]

    x = node_input.astype(jnp.float32)
    attr = node_attr.astype(jnp.float32)
    for i in range(L):
        sc, feat = _node_mm(x, attr, W_sc[i], W_lin1[i])
        parts = _sc_gather_scatter(feat, w_layers[i], src, dst, zeros)
        x = _update(parts, sc, attr, W_lin2[i])
    return x


# speedup vs baseline: 2.9669x; 1.4040x over previous
import functools
import math

import jax
import jax.numpy as jnp
from jax import lax
from jax.experimental import pallas as pl
from jax.experimental.pallas import tpu as pltpu
from jax.experimental.pallas import tpu_sc as plsc

NC = 2
NS = 16
NW = NC * NS
LANES = 16
CHUNK = 64
LOG2 = math.log(2.0)
AVG_DEG = 32.0
BE = 2048
BN = 1000


def _ssp(x):
    return jnp.maximum(x, 0.0) + jnp.log(1.0 + jnp.exp(-jnp.abs(x))) - LOG2


def _edge_w_body(es_ref, ea_ref, w1_ref, w2_ref, out_ref):
    h = jnp.dot(es_ref[...], w1_ref[...], preferred_element_type=jnp.float32)
    w = jnp.dot(_ssp(h), w2_ref[...], preferred_element_type=jnp.float32)
    out_ref[...] = w * ea_ref[...]


def _edge_w(es, ea, W1, W2):
    E_pad, DE = es.shape
    H, D = W2.shape
    return pl.pallas_call(
        _edge_w_body,
        grid=(E_pad // BE,),
        in_specs=[
            pl.BlockSpec((BE, DE), lambda e: (e, 0)),
            pl.BlockSpec((BE, 1), lambda e: (e, 0)),
            pl.BlockSpec((DE, H), lambda e: (0, 0)),
            pl.BlockSpec((H, D), lambda e: (0, 0)),
        ],
        out_specs=pl.BlockSpec((BE, D), lambda e: (e, 0)),
        out_shape=jax.ShapeDtypeStruct((E_pad, D), jnp.float32),
    )(es, ea, W1, W2)


def _node_mm_body(x_ref, attr_ref, wsc_ref, wl1_ref, sc_ref, feat_ref):
    y = x_ref[...] * attr_ref[...]
    sc_ref[...] = jnp.dot(y, wsc_ref[...], preferred_element_type=jnp.float32)
    feat_ref[...] = jnp.dot(y, wl1_ref[...], preferred_element_type=jnp.float32)


def _node_mm(x, attr, Wsc, Wl1):
    N, D = x.shape
    return pl.pallas_call(
        _node_mm_body,
        grid=(N // BN,),
        in_specs=[
            pl.BlockSpec((BN, D), lambda n: (n, 0)),
            pl.BlockSpec((BN, 1), lambda n: (n, 0)),
            pl.BlockSpec((D, D), lambda n: (0, 0)),
            pl.BlockSpec((D, D), lambda n: (0, 0)),
        ],
        out_specs=[
            pl.BlockSpec((BN, D), lambda n: (n, 0)),
            pl.BlockSpec((BN, D), lambda n: (n, 0)),
        ],
        out_shape=[
            jax.ShapeDtypeStruct((N, D), jnp.float32),
            jax.ShapeDtypeStruct((N, D), jnp.float32),
        ],
    )(x, attr, Wsc, Wl1)


def _update_body(p_ref, sc_ref, attr_ref, wl2_ref, x_ref):
    agg = (p_ref[0] + p_ref[1]) * (1.0 / math.sqrt(AVG_DEG))
    conv = jnp.dot(agg * attr_ref[...], wl2_ref[...],
                   preferred_element_type=jnp.float32)
    x_ref[...] = _ssp(sc_ref[...] + conv)


def _update(parts, sc, attr, Wl2):
    N, D = sc.shape
    return pl.pallas_call(
        _update_body,
        grid=(N // BN,),
        in_specs=[
            pl.BlockSpec((NC, BN, D), lambda n: (0, n, 0)),
            pl.BlockSpec((BN, D), lambda n: (n, 0)),
            pl.BlockSpec((BN, 1), lambda n: (n, 0)),
            pl.BlockSpec((D, D), lambda n: (0, 0)),
        ],
        out_specs=pl.BlockSpec((BN, D), lambda n: (n, 0)),
        out_shape=jax.ShapeDtypeStruct((N, D), jnp.float32),
    )(parts, sc, attr, Wl2)


def _sc_gather_scatter(feat, w_e, src, dst, zeros):
    N, D = feat.shape
    E_pad = src.shape[0]
    N_pad = zeros.shape[0]
    EPW = E_pad // NW
    n_chunks = EPW // CHUNK
    RPS = N_pad // NS
    mesh = plsc.VectorSubcoreMesh(core_axis_name="c", subcore_axis_name="s")

    @functools.partial(
        pl.kernel,
        out_type=jax.ShapeDtypeStruct((NC, N_pad, D), jnp.float32),
        mesh=mesh,
        scratch_types=[
            pltpu.VMEM_SHARED((N_pad, D), jnp.float32),
            pltpu.VMEM((CHUNK,), jnp.int32),
            pltpu.VMEM((CHUNK,), jnp.int32),
            pltpu.VMEM((CHUNK,), jnp.int32),
            pltpu.VMEM((CHUNK,), jnp.int32),
            pltpu.VMEM((CHUNK, D), jnp.float32),
            pltpu.VMEM((CHUNK, D), jnp.float32),
            pltpu.VMEM((CHUNK, D), jnp.float32),
            pltpu.VMEM((CHUNK, D), jnp.float32),
            pltpu.SemaphoreType.DMA,
            pltpu.SemaphoreType.DMA,
        ],
    )
    def k(feat_hbm, w_hbm, src_hbm, dst_hbm, zeros_hbm, out_hbm,
          acc_sh, si0, si1, di0, di1, rows0, rows1, w0, w1, g0, g1):
        c = lax.axis_index("c")
        s = lax.axis_index("s")
        wid = s * NC + c
        pltpu.sync_copy(zeros_hbm.at[pl.ds(s * RPS, RPS)],
                        acc_sh.at[pl.ds(s * RPS, RPS)])
        plsc.subcore_barrier()
        base0 = wid * EPW

        def issue(kk, si_b, di_b, rows_b, w_b, sem):
            base = base0 + kk * CHUNK
            pltpu.sync_copy(src_hbm.at[pl.ds(base, CHUNK)], si_b)
            pltpu.sync_copy(dst_hbm.at[pl.ds(base, CHUNK)], di_b)
            pltpu.async_copy(feat_hbm.at[si_b], rows_b, sem)
            pltpu.async_copy(w_hbm.at[pl.ds(base, CHUNK)], w_b, sem)

        def drain(rows_b, w_b, sem):
            pltpu.make_async_copy(feat_hbm.at[pl.ds(0, CHUNK)], rows_b,
                                  sem).wait()
            pltpu.make_async_copy(w_hbm.at[pl.ds(0, CHUNK)], w_b, sem).wait()

        def mul_scatter(di_b, rows_b, w_b):
            def row_body(r, cc):
                for db in range(D // LANES):
                    sl = pl.ds(db * LANES, LANES)
                    rows_b[r, sl] = rows_b[r, sl] * w_b[r, sl]
                return cc

            lax.fori_loop(0, CHUNK, row_body, 0)
            pltpu.sync_copy(rows_b, acc_sh.at[di_b], add=True)

        issue(0, si0, di0, rows0, w0, g0)

        def pair_body(kk2, carry):
            k0 = 2 * kk2
            k1 = k0 + 1

            @pl.when(k1 < n_chunks)
            def _():
                issue(k1, si1, di1, rows1, w1, g1)

            drain(rows0, w0, g0)
            mul_scatter(di0, rows0, w0)

            @pl.when(k1 < n_chunks)
            def _():
                @pl.when(k1 + 1 < n_chunks)
                def _():
                    issue(k1 + 1, si0, di0, rows0, w0, g0)

                drain(rows1, w1, g1)
                mul_scatter(di1, rows1, w1)

            return carry

        lax.fori_loop(0, (n_chunks + 1) // 2, pair_body, 0)
        plsc.subcore_barrier()
        pltpu.sync_copy(acc_sh.at[pl.ds(s * RPS, RPS)],
                        out_hbm.at[c, pl.ds(s * RPS, RPS)])

    return k(feat, w_e, src, dst, zeros)


def kernel(node_input, node_attr, edge_src, edge_dst, edge_attr, edge_scalars,
           W_sc, W_lin1, W_lin2, W_fc1, W_fc2):
    N, D = node_input.shape
    E = edge_src.shape[0]
    L = W_sc.shape[0]
    EPW = ((E + NW - 1) // NW + CHUNK - 1) // CHUNK * CHUNK
    E_pad = EPW * NW
    pe = E_pad - E
    src = jnp.pad(edge_src.astype(jnp.int32), (0, pe))
    dst = jnp.pad(edge_dst.astype(jnp.int32), (0, pe))
    ea = jnp.pad(edge_attr.astype(jnp.float32), ((0, pe), (0, 0)))
    es = jnp.pad(edge_scalars.astype(jnp.float32), ((0, pe), (0, 0)))
    N_pad = (N + 8 * NS - 1) // (8 * NS) * (8 * NS)
    zeros = jnp.zeros((N_pad, D), jnp.float32)

    w_layers = [_edge_w(es, ea, W_fc1[i], W_fc2[i]) for i in range(L)]

    x = node_input.astype(jnp.float32)
    attr = node_attr.astype(jnp.float32)
    for i in range(L):
        sc, feat = _node_mm(x, attr, W_sc[i], W_lin1[i])
        parts = _sc_gather_scatter(feat, w_layers[i], src, dst, zeros)
        x = _update(parts, sc, attr, W_lin2[i])
    return x


# speedup vs baseline: 3.1318x; 1.0556x over previous
import functools
import math

import jax
import jax.numpy as jnp
import numpy as np
from jax import lax
from jax.experimental import pallas as pl
from jax.experimental.pallas import tpu as pltpu
from jax.experimental.pallas import tpu_sc as plsc

NC = 2
NS = 16
NW = NC * NS
LANES = 16
CHUNK = 64
LOG2 = math.log(2.0)
AVG_DEG = 32.0
BE = 2048
BN = 1000


def _ssp(x):
    return jnp.maximum(x, 0.0) + jnp.log(1.0 + jnp.exp(-jnp.abs(x))) - LOG2


def _edge_w_body(es_ref, ea_ref, w1_ref, w2_ref, out_ref):
    h = jnp.dot(es_ref[...], w1_ref[...], preferred_element_type=jnp.float32)
    w = jnp.dot(_ssp(h), w2_ref[...], preferred_element_type=jnp.float32)
    w = w * ea_ref[...]
    D = w.shape[1]
    lo = lax.bitcast_convert_type(w[:, :D // 2].astype(jnp.bfloat16),
                                  jnp.uint16).astype(jnp.int32)
    hi = lax.bitcast_convert_type(w[:, D // 2:].astype(jnp.bfloat16),
                                  jnp.uint16).astype(jnp.int32)
    out_ref[...] = lo | (hi << 16)


def _edge_w(es, ea, W1, W2):
    E_pad, DE = es.shape
    H, D = W2.shape
    return pl.pallas_call(
        _edge_w_body,
        grid=(E_pad // BE,),
        in_specs=[
            pl.BlockSpec((BE, DE), lambda e: (e, 0)),
            pl.BlockSpec((BE, 1), lambda e: (e, 0)),
            pl.BlockSpec((DE, H), lambda e: (0, 0)),
            pl.BlockSpec((H, D), lambda e: (0, 0)),
        ],
        out_specs=pl.BlockSpec((BE, D // 2), lambda e: (e, 0)),
        out_shape=jax.ShapeDtypeStruct((E_pad, D // 2), jnp.int32),
    )(es, ea, W1, W2)


def _node_mm_body(x_ref, attr_ref, wsc_ref, wl1_ref, sc_ref, feat_ref):
    y = x_ref[...] * attr_ref[...]
    sc_ref[...] = jnp.dot(y, wsc_ref[...], preferred_element_type=jnp.float32)
    feat_ref[...] = jnp.dot(y, wl1_ref[...], preferred_element_type=jnp.float32)


def _node_mm(x, attr, Wsc, Wl1):
    N, D = x.shape
    return pl.pallas_call(
        _node_mm_body,
        grid=(N // BN,),
        in_specs=[
            pl.BlockSpec((BN, D), lambda n: (n, 0)),
            pl.BlockSpec((BN, 1), lambda n: (n, 0)),
            pl.BlockSpec((D, D), lambda n: (0, 0)),
            pl.BlockSpec((D, D), lambda n: (0, 0)),
        ],
        out_specs=[
            pl.BlockSpec((BN, D), lambda n: (n, 0)),
            pl.BlockSpec((BN, D), lambda n: (n, 0)),
        ],
        out_shape=[
            jax.ShapeDtypeStruct((N, D), jnp.float32),
            jax.ShapeDtypeStruct((N, D), jnp.float32),
        ],
    )(x, attr, Wsc, Wl1)


def _update_body(p_ref, sc_ref, attr_ref, wl2_ref, x_ref):
    agg = (p_ref[0] + p_ref[1]) * (1.0 / math.sqrt(AVG_DEG))
    conv = jnp.dot(agg * attr_ref[...], wl2_ref[...],
                   preferred_element_type=jnp.float32)
    x_ref[...] = _ssp(sc_ref[...] + conv)


def _update(parts, sc, attr, Wl2):
    N, D = sc.shape
    return pl.pallas_call(
        _update_body,
        grid=(N // BN,),
        in_specs=[
            pl.BlockSpec((NC, BN, D), lambda n: (0, n, 0)),
            pl.BlockSpec((BN, D), lambda n: (n, 0)),
            pl.BlockSpec((BN, 1), lambda n: (n, 0)),
            pl.BlockSpec((D, D), lambda n: (0, 0)),
        ],
        out_specs=pl.BlockSpec((BN, D), lambda n: (n, 0)),
        out_shape=jax.ShapeDtypeStruct((N, D), jnp.float32),
    )(parts, sc, attr, Wl2)


def _sc_gather_scatter(feat, w_e, src, dst, zeros):
    N, D = feat.shape
    E_pad = src.shape[0]
    N_pad = zeros.shape[0]
    EPW = E_pad // NW
    n_chunks = EPW // CHUNK
    RPS = N_pad // NS
    mesh = plsc.VectorSubcoreMesh(core_axis_name="c", subcore_axis_name="s")

    @functools.partial(
        pl.kernel,
        out_type=jax.ShapeDtypeStruct((NC, N_pad, D), jnp.float32),
        mesh=mesh,
        scratch_types=[
            pltpu.VMEM_SHARED((N_pad, D), jnp.float32),
            pltpu.VMEM((CHUNK,), jnp.int32),
            pltpu.VMEM((CHUNK,), jnp.int32),
            pltpu.VMEM((CHUNK,), jnp.int32),
            pltpu.VMEM((CHUNK,), jnp.int32),
            pltpu.VMEM((CHUNK, D), jnp.float32),
            pltpu.VMEM((CHUNK, D), jnp.float32),
            pltpu.VMEM((CHUNK, D // 2), jnp.int32),
            pltpu.VMEM((CHUNK, D // 2), jnp.int32),
            pltpu.SemaphoreType.DMA,
            pltpu.SemaphoreType.DMA,
        ],
    )
    def k(feat_hbm, w_hbm, src_hbm, dst_hbm, zeros_hbm, out_hbm,
          acc_sh, si0, si1, di0, di1, rows0, rows1, w0, w1, g0, g1):
        c = lax.axis_index("c")
        s = lax.axis_index("s")
        wid = s * NC + c
        pltpu.sync_copy(zeros_hbm.at[pl.ds(s * RPS, RPS)],
                        acc_sh.at[pl.ds(s * RPS, RPS)])
        plsc.subcore_barrier()
        base0 = wid * EPW

        def issue(kk, si_b, di_b, rows_b, w_b, sem):
            base = base0 + kk * CHUNK
            pltpu.sync_copy(src_hbm.at[pl.ds(base, CHUNK)], si_b)
            pltpu.async_copy(dst_hbm.at[pl.ds(base, CHUNK)], di_b, sem)
            pltpu.async_copy(feat_hbm.at[si_b], rows_b, sem)
            pltpu.async_copy(w_hbm.at[pl.ds(base, CHUNK)], w_b, sem)

        def drain(di_b, rows_b, w_b, sem):
            pltpu.make_async_copy(dst_hbm.at[pl.ds(0, CHUNK)], di_b,
                                  sem).wait()
            pltpu.make_async_copy(feat_hbm.at[pl.ds(0, CHUNK)], rows_b,
                                  sem).wait()
            pltpu.make_async_copy(w_hbm.at[pl.ds(0, CHUNK)], w_b, sem).wait()

        def mul_scatter(di_b, rows_b, w_b):
            def row_body(r, cc):
                for g in range(D // (2 * LANES)):
                    wv = w_b[r, pl.ds(g * LANES, LANES)]
                    wa = lax.bitcast_convert_type(wv << 16, jnp.float32)
                    wb = lax.bitcast_convert_type(wv & jnp.int32(-65536),
                                                  jnp.float32)
                    sla = pl.ds(g * 2 * LANES, LANES)
                    slb = pl.ds(g * 2 * LANES + LANES, LANES)
                    rows_b[r, sla] = rows_b[r, sla] * wa
                    rows_b[r, slb] = rows_b[r, slb] * wb
                return cc

            lax.fori_loop(0, CHUNK, row_body, 0)
            pltpu.sync_copy(rows_b, acc_sh.at[di_b], add=True)

        issue(0, si0, di0, rows0, w0, g0)

        def pair_body(kk2, carry):
            k0 = 2 * kk2
            k1 = k0 + 1

            @pl.when(k1 < n_chunks)
            def _():
                issue(k1, si1, di1, rows1, w1, g1)

            drain(di0, rows0, w0, g0)
            mul_scatter(di0, rows0, w0)

            @pl.when(k1 < n_chunks)
            def _():
                @pl.when(k1 + 1 < n_chunks)
                def _():
                    issue(k1 + 1, si0, di0, rows0, w0, g0)

                drain(di1, rows1, w1, g1)
                mul_scatter(di1, rows1, w1)

            return carry

        lax.fori_loop(0, (n_chunks + 1) // 2, pair_body, 0)
        plsc.subcore_barrier()
        pltpu.sync_copy(acc_sh.at[pl.ds(s * RPS, RPS)],
                        out_hbm.at[c, pl.ds(s * RPS, RPS)])

    return k(feat, w_e, src, dst, zeros)


def kernel(node_input, node_attr, edge_src, edge_dst, edge_attr, edge_scalars,
           W_sc, W_lin1, W_lin2, W_fc1, W_fc2):
    N, D = node_input.shape
    E = edge_src.shape[0]
    L = W_sc.shape[0]
    EPW = ((E + NW - 1) // NW + CHUNK - 1) // CHUNK * CHUNK
    E_pad = EPW * NW
    pe = E_pad - E
    src = jnp.pad(edge_src.astype(jnp.int32), (0, pe))
    dst = jnp.pad(edge_dst.astype(jnp.int32), (0, pe))
    ea = jnp.pad(edge_attr.astype(jnp.float32), ((0, pe), (0, 0)))
    es = jnp.pad(edge_scalars.astype(jnp.float32), ((0, pe), (0, 0)))
    N_pad = (N + 8 * NS - 1) // (8 * NS) * (8 * NS)
    zeros = jnp.zeros((N_pad, D), jnp.float32)

    perm = jnp.asarray(
        np.arange(D).reshape(D // 32, 2, 16).transpose(1, 0, 2).reshape(D))
    w_layers = [_edge_w(es, ea, W_fc1[i], W_fc2[i][:, perm]) for i in range(L)]

    x = node_input.astype(jnp.float32)
    attr = node_attr.astype(jnp.float32)
    for i in range(L):
        sc, feat = _node_mm(x, attr, W_sc[i], W_lin1[i])
        parts = _sc_gather_scatter(feat, w_layers[i], src, dst, zeros)
        x = _update(parts, sc, attr, W_lin2[i])
    return x


# speedup vs baseline: 3.2702x; 1.0442x over previous
import functools
import math

import jax
import jax.numpy as jnp
import numpy as np
from jax import lax
from jax.experimental import pallas as pl
from jax.experimental.pallas import tpu as pltpu
from jax.experimental.pallas import tpu_sc as plsc

NC = 2
NS = 16
NW = NC * NS
LANES = 16
CHUNK = 64
LOG2 = math.log(2.0)
AVG_DEG = 32.0
BE = 2048
BN = 1000


def _ssp(x):
    return jnp.maximum(x, 0.0) + jnp.log(1.0 + jnp.exp(-jnp.abs(x))) - LOG2


def _edge_w_body(es_ref, ea_ref, w1_ref, w2_ref, out_ref):
    h = jnp.dot(es_ref[...], w1_ref[...], preferred_element_type=jnp.float32)
    w = jnp.dot(_ssp(h), w2_ref[...], preferred_element_type=jnp.float32)
    w = w * ea_ref[...]
    D = w.shape[1]
    lo = lax.bitcast_convert_type(w[:, :D // 2].astype(jnp.bfloat16),
                                  jnp.uint16).astype(jnp.int32)
    hi = lax.bitcast_convert_type(w[:, D // 2:].astype(jnp.bfloat16),
                                  jnp.uint16).astype(jnp.int32)
    out_ref[...] = lo | (hi << 16)


def _edge_w(es, ea, W1, W2):
    E_pad, DE = es.shape
    H, D = W2.shape
    return pl.pallas_call(
        _edge_w_body,
        grid=(E_pad // BE,),
        in_specs=[
            pl.BlockSpec((BE, DE), lambda e: (e, 0)),
            pl.BlockSpec((BE, 1), lambda e: (e, 0)),
            pl.BlockSpec((DE, H), lambda e: (0, 0)),
            pl.BlockSpec((H, D), lambda e: (0, 0)),
        ],
        out_specs=pl.BlockSpec((BE, D // 2), lambda e: (e, 0)),
        out_shape=jax.ShapeDtypeStruct((E_pad, D // 2), jnp.int32),
    )(es, ea, W1, W2)


def _node_mm_body(x_ref, attr_ref, wsc_ref, wl1_ref, sc_ref, feat_ref):
    y = x_ref[...] * attr_ref[...]
    sc_ref[...] = jnp.dot(y, wsc_ref[...], preferred_element_type=jnp.float32)
    feat_ref[...] = jnp.dot(y, wl1_ref[...], preferred_element_type=jnp.float32)


def _node_mm(x, attr, Wsc, Wl1):
    N, D = x.shape
    return pl.pallas_call(
        _node_mm_body,
        grid=(N // BN,),
        in_specs=[
            pl.BlockSpec((BN, D), lambda n: (n, 0)),
            pl.BlockSpec((BN, 1), lambda n: (n, 0)),
            pl.BlockSpec((D, D), lambda n: (0, 0)),
            pl.BlockSpec((D, D), lambda n: (0, 0)),
        ],
        out_specs=[
            pl.BlockSpec((BN, D), lambda n: (n, 0)),
            pl.BlockSpec((BN, D), lambda n: (n, 0)),
        ],
        out_shape=[
            jax.ShapeDtypeStruct((N, D), jnp.float32),
            jax.ShapeDtypeStruct((N, D), jnp.float32),
        ],
    )(x, attr, Wsc, Wl1)


def _update_body(p_ref, sc_ref, attr_ref, wl2_ref, x_ref):
    agg = (p_ref[0] + p_ref[1]) * (1.0 / math.sqrt(AVG_DEG))
    conv = jnp.dot(agg * attr_ref[...], wl2_ref[...],
                   preferred_element_type=jnp.float32)
    x_ref[...] = _ssp(sc_ref[...] + conv)


def _update(parts, sc, attr, Wl2):
    N, D = sc.shape
    return pl.pallas_call(
        _update_body,
        grid=(N // BN,),
        in_specs=[
            pl.BlockSpec((NC, BN, D), lambda n: (0, n, 0)),
            pl.BlockSpec((BN, D), lambda n: (n, 0)),
            pl.BlockSpec((BN, 1), lambda n: (n, 0)),
            pl.BlockSpec((D, D), lambda n: (0, 0)),
        ],
        out_specs=pl.BlockSpec((BN, D), lambda n: (n, 0)),
        out_shape=jax.ShapeDtypeStruct((N, D), jnp.float32),
    )(parts, sc, attr, Wl2)


def _sc_gather_scatter(feat, w_e, src, dst, zeros):
    N, D = feat.shape
    E_pad = src.shape[0]
    N_pad = zeros.shape[0]
    EPW = E_pad // NW
    n_chunks = EPW // CHUNK
    RPS = N_pad // NS
    mesh = plsc.VectorSubcoreMesh(core_axis_name="c", subcore_axis_name="s")

    @functools.partial(
        pl.kernel,
        out_type=jax.ShapeDtypeStruct((NC, N_pad, D), jnp.float32),
        mesh=mesh,
        scratch_types=[
            pltpu.VMEM_SHARED((N_pad, D), jnp.float32),
            pltpu.VMEM((CHUNK,), jnp.int32),
            pltpu.VMEM((CHUNK,), jnp.int32),
            pltpu.VMEM((CHUNK,), jnp.int32),
            pltpu.VMEM((CHUNK,), jnp.int32),
            pltpu.VMEM((CHUNK, D), jnp.float32),
            pltpu.VMEM((CHUNK, D), jnp.float32),
            pltpu.VMEM((CHUNK, D // 2), jnp.int32),
            pltpu.VMEM((CHUNK, D // 2), jnp.int32),
            pltpu.VMEM((CHUNK, D), jnp.float32),
            pltpu.VMEM((CHUNK, D), jnp.float32),
            pltpu.SemaphoreType.DMA,
            pltpu.SemaphoreType.DMA,
            pltpu.SemaphoreType.DMA,
            pltpu.SemaphoreType.DMA,
            pltpu.SemaphoreType.DMA,
            pltpu.SemaphoreType.DMA,
            pltpu.SemaphoreType.DMA,
            pltpu.SemaphoreType.DMA,
        ],
    )
    def k(feat_hbm, w_hbm, src_hbm, dst_hbm, zeros_hbm, out_hbm,
          acc_sh, si0, si1, di0, di1, rows0, rows1, w0, w1, p0, p1,
          i0, i1, d0, d1, g0, g1, s0, s1):
        c = lax.axis_index("c")
        s = lax.axis_index("s")
        wid = s * NC + c
        pltpu.sync_copy(zeros_hbm.at[pl.ds(s * RPS, RPS)],
                        acc_sh.at[pl.ds(s * RPS, RPS)])
        plsc.subcore_barrier()
        base0 = wid * EPW

        SI = (si0, si1)
        DI = (di0, di1)
        ROWS = (rows0, rows1)
        W = (w0, w1)
        P = (p0, p1)
        ISEM = (i0, i1)
        DSEM = (d0, d1)
        GSEM = (g0, g1)
        SSEM = (s0, s1)

        def issue_si(kk, b):
            pltpu.async_copy(src_hbm.at[pl.ds(base0 + kk * CHUNK, CHUNK)],
                             SI[b], ISEM[b])

        def wait_si(b):
            pltpu.make_async_copy(src_hbm.at[pl.ds(0, CHUNK)], SI[b],
                                  ISEM[b]).wait()

        def issue_di(kk, b):
            pltpu.async_copy(dst_hbm.at[pl.ds(base0 + kk * CHUNK, CHUNK)],
                             DI[b], DSEM[b])

        def wait_di(b):
            pltpu.make_async_copy(dst_hbm.at[pl.ds(0, CHUNK)], DI[b],
                                  DSEM[b]).wait()

        def issue_gather(kk, b):
            pltpu.async_copy(feat_hbm.at[SI[b]], ROWS[b], GSEM[b])
            pltpu.async_copy(w_hbm.at[pl.ds(base0 + kk * CHUNK, CHUNK)],
                             W[b], GSEM[b])

        def wait_gather(b):
            pltpu.make_async_copy(feat_hbm.at[pl.ds(0, CHUNK)], ROWS[b],
                                  GSEM[b]).wait()
            pltpu.make_async_copy(w_hbm.at[pl.ds(0, CHUNK)], W[b],
                                  GSEM[b]).wait()

        def issue_scatter(b):
            pltpu.async_copy(P[b], acc_sh.at[DI[b]], SSEM[b], add=True)

        def wait_scatter(b):
            pltpu.make_async_copy(P[b], acc_sh.at[DI[b]], SSEM[b]).wait()

        def mul(b):
            rows_b, w_b, p_b = ROWS[b], W[b], P[b]

            def row_body(r, cc):
                for g in range(D // (2 * LANES)):
                    wv = w_b[r, pl.ds(g * LANES, LANES)]
                    wa = lax.bitcast_convert_type(wv << 16, jnp.float32)
                    wb = lax.bitcast_convert_type(wv & jnp.int32(-65536),
                                                  jnp.float32)
                    sla = pl.ds(g * 2 * LANES, LANES)
                    slb = pl.ds(g * 2 * LANES + LANES, LANES)
                    p_b[r, sla] = rows_b[r, sla] * wa
                    p_b[r, slb] = rows_b[r, slb] * wb
                return cc

            lax.fori_loop(0, CHUNK, row_body, 0)

        pltpu.sync_copy(src_hbm.at[pl.ds(base0, CHUNK)], si0)
        pltpu.sync_copy(src_hbm.at[pl.ds(base0 + CHUNK, CHUNK)], si1)
        issue_di(0, 0)
        issue_di(1, 1)
        issue_gather(0, 0)

        def pair_body(t, carry):
            for b in (0, 1):
                kk = 2 * t + b

                @pl.when(kk < n_chunks)
                def _():
                    @pl.when(kk + 1 < n_chunks)
                    def _():
                        issue_gather(kk + 1, 1 - b)

                    @pl.when(kk + 2 < n_chunks)
                    def _():
                        issue_si(kk + 2, b)

                    wait_gather(b)

                    @pl.when(kk >= 2)
                    def _():
                        wait_scatter(b)
                        issue_di(kk, b)

                    mul(b)
                    wait_di(b)
                    issue_scatter(b)

                    @pl.when(kk + 2 < n_chunks)
                    def _():
                        wait_si(b)

            return carry

        lax.fori_loop(0, (n_chunks + 1) // 2, pair_body, 0)
        wait_scatter((n_chunks - 2) % 2)
        wait_scatter((n_chunks - 1) % 2)
        plsc.subcore_barrier()
        pltpu.sync_copy(acc_sh.at[pl.ds(s * RPS, RPS)],
                        out_hbm.at[c, pl.ds(s * RPS, RPS)])

    return k(feat, w_e, src, dst, zeros)


def kernel(node_input, node_attr, edge_src, edge_dst, edge_attr, edge_scalars,
           W_sc, W_lin1, W_lin2, W_fc1, W_fc2):
    N, D = node_input.shape
    E = edge_src.shape[0]
    L = W_sc.shape[0]
    EPW = ((E + NW - 1) // NW + CHUNK - 1) // CHUNK * CHUNK
    E_pad = EPW * NW
    pe = E_pad - E
    src = jnp.pad(edge_src.astype(jnp.int32), (0, pe))
    dst = jnp.pad(edge_dst.astype(jnp.int32), (0, pe))
    ea = jnp.pad(edge_attr.astype(jnp.float32), ((0, pe), (0, 0)))
    es = jnp.pad(edge_scalars.astype(jnp.float32), ((0, pe), (0, 0)))
    N_pad = (N + 8 * NS - 1) // (8 * NS) * (8 * NS)
    zeros = jnp.zeros((N_pad, D), jnp.float32)

    perm = jnp.asarray(
        np.arange(D).reshape(D // 32, 2, 16).transpose(1, 0, 2).reshape(D))
    w_layers = [_edge_w(es, ea, W_fc1[i], W_fc2[i][:, perm]) for i in range(L)]

    x = node_input.astype(jnp.float32)
    attr = node_attr.astype(jnp.float32)
    for i in range(L):
        sc, feat = _node_mm(x, attr, W_sc[i], W_lin1[i])
        parts = _sc_gather_scatter(feat, w_layers[i], src, dst, zeros)
        x = _update(parts, sc, attr, W_lin2[i])
    return x
